# baseline (speedup 1.0000x reference)
import functools

import jax
import jax.numpy as jnp
from jax import lax
from jax.experimental import pallas as pl
from jax.experimental.pallas import tpu as pltpu
from jax.experimental.pallas import tpu_sc as plsc

_B = 16384
_D = 64
_NSCS = 2
_BPS = _B // _NSCS
_ICH = 1024
_NICH = _BPS // _ICH


def _make_scs_gather():
    mesh = plsc.ScalarSubcoreMesh(axis_name="c", num_cores=_NSCS)

    @functools.partial(
        pl.kernel,
        mesh=mesh,
        out_type=[
            jax.ShapeDtypeStruct((_B, _D), jnp.float32),
            jax.ShapeDtypeStruct((_B, _D), jnp.float32),
        ],
        scratch_types=[
            pltpu.SMEM((_ICH,), jnp.int32),
            pltpu.SMEM((_ICH,), jnp.int32),
            pltpu.SemaphoreType.DMA,
            pltpu.SemaphoreType.DMA,
            pltpu.SemaphoreType.DMA,
        ],
    )
    def gather_k(pidx_hbm, tidx_hbm, pemb_hbm, temb_hbm, pout_hbm, tout_hbm,
                 pidx_s, tidx_s, sem_i, sem_p, sem_t):
        cid = lax.axis_index("c")
        base = cid * _BPS

        def chunk(ci, _):
            cbase = base + ci * _ICH
            pltpu.async_copy(pidx_hbm.at[pl.ds(cbase, _ICH)], pidx_s, sem_i).wait()
            pltpu.async_copy(tidx_hbm.at[pl.ds(cbase, _ICH)], tidx_s, sem_i).wait()

            def row(r, _):
                i = pidx_s[r]
                j = tidx_s[r]
                k = cbase + r
                pltpu.async_copy(pemb_hbm.at[pl.ds(i, 1)],
                                 pout_hbm.at[pl.ds(k, 1)], sem_p)
                pltpu.async_copy(temb_hbm.at[pl.ds(j, 1)],
                                 tout_hbm.at[pl.ds(k, 1)], sem_t)
                return 0

            lax.fori_loop(0, _ICH, row, 0)

            def drain(r, _):
                k = cbase + r
                pltpu.make_async_copy(pemb_hbm.at[pl.ds(0, 1)],
                                      pout_hbm.at[pl.ds(k, 1)], sem_p).wait()
                pltpu.make_async_copy(temb_hbm.at[pl.ds(0, 1)],
                                      tout_hbm.at[pl.ds(k, 1)], sem_t).wait()
                return 0

            lax.fori_loop(0, _ICH, drain, 0)
            return 0

        lax.fori_loop(0, _NICH, chunk, 0)

    return gather_k


_scs_gather = _make_scs_gather()

_TC_BLK = 2048


def _tc_body(p_ref, t_ref, wp_ref, bp_ref, wt_ref, bt_ref, o_ref):
    dn = (((1,), (1,)), ((), ()))
    ph = jnp.maximum(
        lax.dot_general(p_ref[...], wp_ref[...], dn,
                        preferred_element_type=jnp.float32) + bp_ref[...], 0.0)
    th = jnp.maximum(
        lax.dot_general(t_ref[...], wt_ref[...], dn,
                        preferred_element_type=jnp.float32) + bt_ref[...], 0.0)
    o_ref[...] = jnp.sum(ph * th, axis=1, keepdims=True)


def _tc_score(p_rows, t_rows, W_p, b_p, W_t, b_t):
    grid = (_B // _TC_BLK,)
    return pl.pallas_call(
        _tc_body,
        grid=grid,
        in_specs=[
            pl.BlockSpec((_TC_BLK, _D), lambda i: (i, 0)),
            pl.BlockSpec((_TC_BLK, _D), lambda i: (i, 0)),
            pl.BlockSpec((_D, _D), lambda i: (0, 0)),
            pl.BlockSpec((1, _D), lambda i: (0, 0)),
            pl.BlockSpec((_D, _D), lambda i: (0, 0)),
            pl.BlockSpec((1, _D), lambda i: (0, 0)),
        ],
        out_specs=pl.BlockSpec((_TC_BLK, 1), lambda i: (i, 0)),
        out_shape=jax.ShapeDtypeStruct((_B, 1), jnp.float32),
    )(p_rows, t_rows, W_p, b_p.reshape(1, _D), W_t, b_t.reshape(1, _D))


def kernel(p_idx, t_idx, play_emb, track_emb, W_p, b_p, W_t, b_t):
    p_rows, t_rows = _scs_gather(p_idx.astype(jnp.int32), t_idx.astype(jnp.int32),
                                 play_emb, track_emb)
    out = _tc_score(p_rows, t_rows, W_p, b_p, W_t, b_t)
    return out[:, 0]


# speedup vs baseline: 1.0136x; 1.0136x over previous
import functools

import jax
import jax.numpy as jnp
from jax import lax
from jax.experimental import pallas as pl
from jax.experimental.pallas import tpu as pltpu
from jax.experimental.pallas import tpu_sc as plsc

_B = 16384
_D = 64
_NC = 2
_NS = 16
_NW = _NC * _NS
_BPW = _B // _NW
_L = 16
_NVREG = _BPW // _L


def _make_gather():
    mesh = plsc.VectorSubcoreMesh(core_axis_name="c", subcore_axis_name="s")

    @functools.partial(
        pl.kernel,
        mesh=mesh,
        compiler_params=pltpu.CompilerParams(needs_layout_passes=False),
        out_type=[
            jax.ShapeDtypeStruct((_B, _D), jnp.float32),
            jax.ShapeDtypeStruct((_B, _D), jnp.float32),
        ],
        scratch_types=[
            pltpu.VMEM((_BPW,), jnp.int32),
            pltpu.VMEM((_BPW,), jnp.int32),
            pltpu.SemaphoreType.DMA,
            pltpu.SemaphoreType.DMA,
            pltpu.SemaphoreType.DMA,
        ],
    )
    def gather_k(pidx_hbm, tidx_hbm, pemb_hbm, temb_hbm, pout_hbm, tout_hbm,
                 pidx_v, tidx_v, sem_i, sem_p, sem_t):
        wid = lax.axis_index("s") * _NC + lax.axis_index("c")
        base = wid * _BPW
        pltpu.async_copy(pidx_hbm.at[pl.ds(base, _BPW)], pidx_v, sem_i).wait()
        pltpu.async_copy(tidx_hbm.at[pl.ds(base, _BPW)], tidx_v, sem_i).wait()

        lane = lax.iota(jnp.int32, _L)

        def chunk(v, _):
            pv = pidx_v[pl.ds(v * _L, _L)]
            tv = tidx_v[pl.ds(v * _L, _L)]
            for l in range(_L):
                i = lax.reduce_max(jnp.where(lane == l, pv, 0), axes=(0,))
                j = lax.reduce_max(jnp.where(lane == l, tv, 0), axes=(0,))
                k = base + v * _L + l
                pltpu.async_copy(pemb_hbm.at[pl.ds(i, 1)],
                                 pout_hbm.at[pl.ds(k, 1)], sem_p)
                pltpu.async_copy(temb_hbm.at[pl.ds(j, 1)],
                                 tout_hbm.at[pl.ds(k, 1)], sem_t)
            return 0

        lax.fori_loop(0, _NVREG, chunk, 0)

        pltpu.make_async_copy(pemb_hbm.at[pl.ds(0, _BPW)],
                              pout_hbm.at[pl.ds(base, _BPW)], sem_p).wait()
        pltpu.make_async_copy(temb_hbm.at[pl.ds(0, _BPW)],
                              tout_hbm.at[pl.ds(base, _BPW)], sem_t).wait()

    return gather_k


_sc_gather = _make_gather()

_TC_BLK = 2048


def _tc_body(p_ref, t_ref, wp_ref, bp_ref, wt_ref, bt_ref, o_ref):
    dn = (((1,), (1,)), ((), ()))
    ph = jnp.maximum(
        lax.dot_general(p_ref[...], wp_ref[...], dn,
                        preferred_element_type=jnp.float32) + bp_ref[...], 0.0)
    th = jnp.maximum(
        lax.dot_general(t_ref[...], wt_ref[...], dn,
                        preferred_element_type=jnp.float32) + bt_ref[...], 0.0)
    o_ref[...] = jnp.sum(ph * th, axis=1, keepdims=True)


def _tc_score(p_rows, t_rows, W_p, b_p, W_t, b_t):
    grid = (_B // _TC_BLK,)
    return pl.pallas_call(
        _tc_body,
        grid=grid,
        in_specs=[
            pl.BlockSpec((_TC_BLK, _D), lambda i: (i, 0)),
            pl.BlockSpec((_TC_BLK, _D), lambda i: (i, 0)),
            pl.BlockSpec((_D, _D), lambda i: (0, 0)),
            pl.BlockSpec((1, _D), lambda i: (0, 0)),
            pl.BlockSpec((_D, _D), lambda i: (0, 0)),
            pl.BlockSpec((1, _D), lambda i: (0, 0)),
        ],
        out_specs=pl.BlockSpec((_TC_BLK, 1), lambda i: (i, 0)),
        out_shape=jax.ShapeDtypeStruct((_B, 1), jnp.float32),
    )(p_rows, t_rows, W_p, b_p.reshape(1, _D), W_t, b_t.reshape(1, _D))


def kernel(p_idx, t_idx, play_emb, track_emb, W_p, b_p, W_t, b_t):
    p_rows, t_rows = _sc_gather(p_idx.astype(jnp.int32), t_idx.astype(jnp.int32),
                                play_emb, track_emb)
    out = _tc_score(p_rows, t_rows, W_p, b_p, W_t, b_t)
    return out[:, 0]


# speedup vs baseline: 1.3401x; 1.3221x over previous
import functools

import jax
import jax.numpy as jnp
from jax import lax
from jax.experimental import pallas as pl
from jax.experimental.pallas import tpu as pltpu
from jax.experimental.pallas import tpu_sc as plsc

_B = 16384
_D = 64
_NC = 2
_NS = 16
_NW = _NC * _NS
_BPW = _B // _NW
_CH = 128
_NCHUNK = _BPW // _CH

_TC_BLK = 2048


def _make_sc_gather():
    mesh = plsc.VectorSubcoreMesh(core_axis_name="c", subcore_axis_name="s")

    @functools.partial(
        pl.kernel,
        mesh=mesh,
        compiler_params=pltpu.CompilerParams(use_tc_tiling_on_sc=False, skip_device_barrier=True),
        out_type=[
            jax.ShapeDtypeStruct((_B, _D), jnp.float32),
            jax.ShapeDtypeStruct((_B, _D), jnp.float32),
        ],
        scratch_types=[
            pltpu.VMEM((_NCHUNK, _CH), jnp.int32),
            pltpu.VMEM((_NCHUNK, _CH), jnp.int32),
            pltpu.VMEM((_BPW, _D), jnp.float32),
            pltpu.VMEM((_BPW, _D), jnp.float32),
            pltpu.SemaphoreType.DMA,
        ],
    )
    def gather_k(pidx_hbm, tidx_hbm, pemb_hbm, temb_hbm, pout_hbm, tout_hbm,
                 pidx_v, tidx_v, prow_v, trow_v, sem):
        wid = lax.axis_index("s") * _NC + lax.axis_index("c")
        base = wid * _BPW
        pltpu.sync_copy(pidx_hbm.at[wid], pidx_v)
        pltpu.sync_copy(tidx_hbm.at[wid], tidx_v)
        copies = []
        for j in range(_NCHUNK):
            copies.append(pltpu.async_copy(
                pemb_hbm.at[pidx_v.at[j]], prow_v.at[pl.ds(j * _CH, _CH)], sem))
            copies.append(pltpu.async_copy(
                temb_hbm.at[tidx_v.at[j]], trow_v.at[pl.ds(j * _CH, _CH)], sem))
        for c in copies:
            c.wait()
        pltpu.sync_copy(prow_v, pout_hbm.at[pl.ds(base, _BPW)])
        pltpu.sync_copy(trow_v, tout_hbm.at[pl.ds(base, _BPW)])

    return gather_k


_sc_gather = _make_sc_gather()


def _tc_body(p_ref, t_ref, wp_ref, bp_ref, wt_ref, bt_ref, o_ref):
    dn = (((1,), (1,)), ((), ()))
    ph = jnp.maximum(
        lax.dot_general(p_ref[...], wp_ref[...], dn,
                        preferred_element_type=jnp.float32) + bp_ref[...], 0.0)
    th = jnp.maximum(
        lax.dot_general(t_ref[...], wt_ref[...], dn,
                        preferred_element_type=jnp.float32) + bt_ref[...], 0.0)
    o_ref[...] = jnp.sum(ph * th, axis=1, keepdims=True)


def _tc_score(p_rows, t_rows, W_p, b_p, W_t, b_t):
    grid = (_B // _TC_BLK,)
    return pl.pallas_call(
        _tc_body,
        grid=grid,
        in_specs=[
            pl.BlockSpec((_TC_BLK, _D), lambda i: (i, 0)),
            pl.BlockSpec((_TC_BLK, _D), lambda i: (i, 0)),
            pl.BlockSpec((_D, _D), lambda i: (0, 0)),
            pl.BlockSpec((1, _D), lambda i: (0, 0)),
            pl.BlockSpec((_D, _D), lambda i: (0, 0)),
            pl.BlockSpec((1, _D), lambda i: (0, 0)),
        ],
        out_specs=pl.BlockSpec((_TC_BLK, 1), lambda i: (i, 0)),
        out_shape=jax.ShapeDtypeStruct((_B, 1), jnp.float32),
    )(p_rows, t_rows, W_p, b_p.reshape(1, _D), W_t, b_t.reshape(1, _D))


def kernel(p_idx, t_idx, play_emb, track_emb, W_p, b_p, W_t, b_t):
    pidx3 = p_idx.astype(jnp.int32).reshape(_NW, _NCHUNK, _CH)
    tidx3 = t_idx.astype(jnp.int32).reshape(_NW, _NCHUNK, _CH)
    p_rows, t_rows = _sc_gather(pidx3, tidx3, play_emb, track_emb)
    out = _tc_score(p_rows, t_rows, W_p, b_p, W_t, b_t)
    return out[:, 0]
